# baseline (speedup 1.0000x reference)
import functools

import jax
import jax.numpy as jnp
from jax import lax
from jax.experimental import pallas as pl
from jax.experimental.pallas import tpu as pltpu
from jax.experimental.pallas import tpu_sc as plsc

BATCH = 16384
EMBED_DIM = 32
NROWS = 1000000
L = 16
NC = 2
NS = 16
NW = NC * NS
BPW = BATCH // NW
CHUNK = 128
NCHUNK = BPW // CHUNK
NBUF = 2
NG = CHUNK // L


def _mesh():
    return plsc.VectorSubcoreMesh(core_axis_name="c", subcore_axis_name="s")


@functools.partial(
    pl.kernel,
    out_type=jax.ShapeDtypeStruct((BATCH,), jnp.float32),
    mesh=_mesh(),
    scratch_types=[
        pltpu.VMEM((BPW + L,), jnp.int32),
        pltpu.VMEM((BPW + L,), jnp.int32),
        pltpu.VMEM((NBUF, CHUNK, EMBED_DIM), jnp.float32),
        pltpu.VMEM((NBUF, CHUNK, EMBED_DIM), jnp.float32),
        pltpu.VMEM((2 * EMBED_DIM, L), jnp.float32),
        pltpu.VMEM((L,), jnp.float32),
        pltpu.VMEM((BPW,), jnp.float32),
        pltpu.SemaphoreType.DMA,
        pltpu.SemaphoreType.DMA,
    ],
    compiler_params=pltpu.CompilerParams(needs_layout_passes=False),
)
def _sc_kernel(users_hbm, movies_hbm, ut_hbm, mt_hbm, wb_hbm, bv_hbm, out_hbm,
               uidx_v, midx_v, urows_v, mrows_v, wb_v, bv_v, acc_v,
               usem, msem):
    wid = lax.axis_index("s") * NC + lax.axis_index("c")
    base = wid * BPW

    pltpu.sync_copy(users_hbm.at[pl.ds(base, BPW)], uidx_v.at[pl.ds(0, BPW)])
    pltpu.sync_copy(movies_hbm.at[pl.ds(base, BPW)], midx_v.at[pl.ds(0, BPW)])
    pltpu.sync_copy(wb_hbm, wb_v)
    pltpu.sync_copy(bv_hbm, bv_v)

    def fire(j):
        slot = j % NBUF

        def en(i, _, j=j, slot=slot):
            g = j * CHUNK + i
            ru = uidx_v[pl.ds(g, L)][0]
            rm = midx_v[pl.ds(g, L)][0]
            pltpu.async_copy(ut_hbm.at[ru], urows_v.at[slot, i], usem)
            pltpu.async_copy(mt_hbm.at[rm], mrows_v.at[slot, i], msem)
            return 0

        lax.fori_loop(0, CHUNK, en, 0, unroll=2)

    def drain(slot):
        def wt(i, _, slot=slot):
            pltpu.make_async_copy(ut_hbm.at[0], urows_v.at[slot, i], usem).wait()
            pltpu.make_async_copy(mt_hbm.at[0], mrows_v.at[slot, i], msem).wait()
            return 0

        lax.fori_loop(0, CHUNK, wt, 0, unroll=2)

    fire(0)
    if NCHUNK > 1:
        fire(1)

    bias = bv_v[...]
    lanes = lax.iota(jnp.int32, L)
    slotv = [jnp.full((L,), s, jnp.int32) for s in range(NBUF)]
    ivecs = [lanes + k * L for k in range(NG)]

    for j in range(NCHUNK):
        slot = j % NBUF
        drain(slot)

        def col(c, accs, slot=slot):
            cv = jnp.full((L,), c, jnp.int32)
            wu = wb_v[c, pl.ds(0, L)]
            wm = wb_v[c + EMBED_DIM, pl.ds(0, L)]
            out = []
            for k in range(NG):
                u = plsc.load_gather(urows_v, [slotv[slot], ivecs[k], cv])
                m = plsc.load_gather(mrows_v, [slotv[slot], ivecs[k], cv])
                out.append(accs[k] + u * wu + m * wm)
            return tuple(out)

        accs = lax.fori_loop(0, EMBED_DIM, col, tuple([bias] * NG))
        for k in range(NG):
            acc_v[pl.ds(j * CHUNK + k * L, L)] = accs[k]
        if j + NBUF < NCHUNK:
            fire(j + NBUF)

    pltpu.sync_copy(acc_v, out_hbm.at[pl.ds(base, BPW)])


def kernel(users, movies, user_table, movie_table, W, b):
    w_flat = W.reshape(2 * EMBED_DIM).astype(jnp.float32)
    wb = jnp.broadcast_to(w_flat[:, None], (2 * EMBED_DIM, L))
    bv = jnp.full((L,), b[0], dtype=jnp.float32)
    out = _sc_kernel(users.astype(jnp.int32), movies.astype(jnp.int32),
                     user_table, movie_table, wb, bv)
    return out.reshape(BATCH, 1)
